# baseline (speedup 1.0000x reference)
import functools

import jax
import jax.numpy as jnp
from jax import lax
from jax.experimental import pallas as pl
from jax.experimental.pallas import tpu as pltpu
from jax.experimental.pallas import tpu_sc as plsc

_REG = 0.01
_B = 4096
_D = 64
_LANES = 16
_NC = 2
_NS = 16
_NW = _NC * _NS
_BPW = _B // _NW
_N = 100000
_TBLK = 4096
_TGRID = -(-_N // _TBLK)
_HB = _TBLK // 2
_SH = _TBLK.bit_length() - 1
_HMASK = _HB - 1


def _tr_body(inu_ref, inv_ref, outu_ref, outv_ref):
    e = jnp.eye(_D, dtype=jnp.float32)
    for in_ref, out_ref in ((inu_ref, outu_ref), (inv_ref, outv_ref)):
        x = in_ref[...]
        t1 = jnp.transpose(x[:, :_HB], (1, 0))
        t2 = lax.dot_general(x[:, _HB:], e, (((0,), (0,)), ((), ())),
                             preferred_element_type=jnp.float32)
        out_ref[...] = jnp.concatenate([t1, t2], axis=1)


def _pack_tables(tu, tv):
    return pl.pallas_call(
        _tr_body,
        grid=(_TGRID,),
        in_specs=[pl.BlockSpec((_D, _TBLK), lambda b: (0, b)),
                  pl.BlockSpec((_D, _TBLK), lambda b: (0, b))],
        out_specs=[pl.BlockSpec((_HB, 2 * _D), lambda b: (b, 0)),
                   pl.BlockSpec((_HB, 2 * _D), lambda b: (b, 0))],
        out_shape=[jax.ShapeDtypeStruct((_TGRID * _HB, 2 * _D), jnp.float32),
                   jax.ShapeDtypeStruct((_TGRID * _HB, 2 * _D), jnp.float32)],
    )(tu, tv)


def _sc_body(u_hbm, i_hbm, j_hbm, Up_hbm, Vp_hbm, part_hbm, sq_hbm,
             u_v, i_v, j_v, hu_v, hi_v, hj_v, gu_v, gi_v, gj_v,
             du, di, dj, part_v, sq_v, s0, s1, s2):
    wid = lax.axis_index("s") * _NC + lax.axis_index("c")
    base = wid * _BPW
    pltpu.sync_copy(u_hbm.at[pl.ds(base, _BPW)], u_v)
    pltpu.sync_copy(i_hbm.at[pl.ds(base, _BPW)], i_v)
    pltpu.sync_copy(j_hbm.at[pl.ds(base, _BPW)], j_v)
    for q in range(_BPW // _LANES):
        sl = pl.ds(q * _LANES, _LANES)
        for sv, dst, hb in ((u_v, gu_v, hu_v), (i_v, gi_v, hi_v), (j_v, gj_v, hj_v)):
            r = sv[sl]
            dst[sl] = jnp.left_shift(jnp.right_shift(r, _SH), _SH - 1) + jnp.bitwise_and(r, _HMASK)
            hb[sl] = jnp.bitwise_and(jnp.right_shift(r, _SH - 1), 1) * _D

    cu = pltpu.async_copy(Up_hbm.at[gu_v], du, s0)
    ci = pltpu.async_copy(Vp_hbm.at[gi_v], di, s1)
    cj = pltpu.async_copy(Vp_hbm.at[gj_v], dj, s2)
    cu.wait()
    ci.wait()
    cj.wait()

    rows = [lax.iota(jnp.int32, _LANES) + q * _LANES for q in range(_D // _LANES)]


    lane = lax.iota(jnp.int32, _LANES)

    def chunk(c, sq):
        cb = c * _LANES
        hu16 = hu_v[pl.ds(cb, _LANES)]
        hi16 = hi_v[pl.ds(cb, _LANES)]
        hj16 = hj_v[pl.ds(cb, _LANES)]
        for t in range(_LANES):
            b = cb + t
            hu = hu16[t]
            hi = hi16[t]
            hj = hj16[t]
            acc = jnp.zeros((_LANES,), jnp.float32)
            for q in range(_D // _LANES):
                fu = du[b, pl.ds(hu + q * _LANES, _LANES)]
                fi_ = di[b, pl.ds(hi + q * _LANES, _LANES)]
                fj_ = dj[b, pl.ds(hj + q * _LANES, _LANES)]
                acc = acc + fu * (fi_ - fj_)
                sq = sq + fu * fu + fi_ * fi_ + fj_ * fj_
            part_v[b, :] = acc
        return sq

    sq = lax.fori_loop(0, _BPW // _LANES, chunk, jnp.zeros((_LANES,), jnp.float32))
    sq_v[...] = sq
    pltpu.sync_copy(part_v, part_hbm.at[pl.ds(base, _BPW)])
    pltpu.sync_copy(sq_v, sq_hbm.at[pl.ds(wid * _LANES, _LANES)])


_sc_call = functools.partial(
    pl.kernel,
    out_type=(
        jax.ShapeDtypeStruct((_B, _LANES), jnp.float32),
        jax.ShapeDtypeStruct((_NW * _LANES,), jnp.float32),
    ),
    mesh=plsc.VectorSubcoreMesh(core_axis_name="c", subcore_axis_name="s"),
    scratch_types=[
        pltpu.VMEM((_BPW,), jnp.int32),
        pltpu.VMEM((_BPW,), jnp.int32),
        pltpu.VMEM((_BPW,), jnp.int32),
        pltpu.VMEM((_BPW,), jnp.int32),
        pltpu.VMEM((_BPW,), jnp.int32),
        pltpu.VMEM((_BPW,), jnp.int32),
        pltpu.VMEM((_BPW,), jnp.int32),
        pltpu.VMEM((_BPW,), jnp.int32),
        pltpu.VMEM((_BPW,), jnp.int32),
        pltpu.VMEM((_BPW, 2 * _D), jnp.float32),
        pltpu.VMEM((_BPW, 2 * _D), jnp.float32),
        pltpu.VMEM((_BPW, 2 * _D), jnp.float32),
        pltpu.VMEM((_BPW, _LANES), jnp.float32),
        pltpu.VMEM((_LANES,), jnp.float32),
        pltpu.SemaphoreType.DMA,
        pltpu.SemaphoreType.DMA,
        pltpu.SemaphoreType.DMA,
    ],
)(_sc_body)


def _tc_body(part_ref, sq_ref, out_ref):
    x = jnp.sum(part_ref[...], axis=1, keepdims=True)
    s = jnp.sum(jnp.log2(jax.nn.sigmoid(x)))
    ssq = jnp.sum(sq_ref[...])
    out_ref[0, 0] = _REG * ssq - s


def kernel(u, i, j, U, V):
    u = u.astype(jnp.int32)
    i = i.astype(jnp.int32)
    j = j.astype(jnp.int32)
    Up, Vp = _pack_tables(U.T, V.T)
    part, sq = _sc_call(u, i, j, Up, Vp)
    loss = pl.pallas_call(
        _tc_body,
        out_shape=jax.ShapeDtypeStruct((1, 1), jnp.float32),
        in_specs=[
            pl.BlockSpec(memory_space=pltpu.VMEM),
            pl.BlockSpec(memory_space=pltpu.VMEM),
        ],
        out_specs=pl.BlockSpec(memory_space=pltpu.SMEM),
    )(part, sq.reshape(4, _NW * _LANES // 4))
    return loss[0, 0]


# speedup vs baseline: 1.0667x; 1.0667x over previous
import functools

import jax
import jax.numpy as jnp
from jax import lax
from jax.experimental import pallas as pl
from jax.experimental.pallas import tpu as pltpu
from jax.experimental.pallas import tpu_sc as plsc

_REG = 0.01
_B = 4096
_D = 64
_LANES = 16
_NC = 2
_NS = 16
_NW = _NC * _NS
_BPW = _B // _NW
_N = 100000
_TBLK = 8192
_TGRID = -(-_N // _TBLK)
_HB = _TBLK // 2
_SH = _TBLK.bit_length() - 1
_HMASK = _HB - 1


def _tr_body(inu_ref, inv_ref, outu_ref, outv_ref):
    e = jnp.eye(_D, dtype=jnp.float32)
    for in_ref, out_ref in ((inu_ref, outu_ref), (inv_ref, outv_ref)):
        x = in_ref[...]
        t1 = jnp.transpose(x[:, :_HB], (1, 0))
        t2 = lax.dot_general(x[:, _HB:], e, (((0,), (0,)), ((), ())),
                             preferred_element_type=jnp.float32)
        out_ref[...] = jnp.concatenate([t1, t2], axis=1)


def _pack_tables(tu, tv):
    return pl.pallas_call(
        _tr_body,
        grid=(_TGRID,),
        in_specs=[pl.BlockSpec((_D, _TBLK), lambda b: (0, b)),
                  pl.BlockSpec((_D, _TBLK), lambda b: (0, b))],
        out_specs=[pl.BlockSpec((_HB, 2 * _D), lambda b: (b, 0)),
                   pl.BlockSpec((_HB, 2 * _D), lambda b: (b, 0))],
        out_shape=[jax.ShapeDtypeStruct((_TGRID * _HB, 2 * _D), jnp.float32),
                   jax.ShapeDtypeStruct((_TGRID * _HB, 2 * _D), jnp.float32)],
    )(tu, tv)


def _sc_body(u_hbm, i_hbm, j_hbm, Up_hbm, Vp_hbm, part_hbm, sq_hbm,
             u_v, i_v, j_v, hu_v, hi_v, hj_v,
             gu0_v, gu1_v, gi0_v, gi1_v, gj0_v, gj1_v,
             du, di, dj, part_v, sq_v, s0, s1, s2, s3, s4, s5):
    wid = lax.axis_index("s") * _NC + lax.axis_index("c")
    base = wid * _BPW
    c0 = pltpu.async_copy(u_hbm.at[pl.ds(base, _BPW)], u_v, s0)
    c1 = pltpu.async_copy(i_hbm.at[pl.ds(base, _BPW)], i_v, s1)
    c2 = pltpu.async_copy(j_hbm.at[pl.ds(base, _BPW)], j_v, s2)
    c0.wait()
    c1.wait()
    c2.wait()
    nq = _BPW // _LANES
    for q in range(nq):
        sl = pl.ds(q * _LANES, _LANES)
        hsl = pl.ds((q % (nq // 2)) * _LANES, _LANES)
        for sv, d0, d1, hb in ((u_v, gu0_v, gu1_v, hu_v),
                               (i_v, gi0_v, gi1_v, hi_v),
                               (j_v, gj0_v, gj1_v, hj_v)):
            r = sv[sl]
            g = jnp.left_shift(jnp.right_shift(r, _SH), _SH - 1) + jnp.bitwise_and(r, _HMASK)
            if q < nq // 2:
                d0[hsl] = g
            else:
                d1[hsl] = g
            hb[sl] = jnp.bitwise_and(jnp.right_shift(r, _SH - 1), 1) * _D

    half = _BPW // 2
    ga = [pltpu.async_copy(Up_hbm.at[gu0_v], du.at[pl.ds(0, half)], s0),
          pltpu.async_copy(Vp_hbm.at[gi0_v], di.at[pl.ds(0, half)], s1),
          pltpu.async_copy(Vp_hbm.at[gj0_v], dj.at[pl.ds(0, half)], s2)]
    gb = [pltpu.async_copy(Up_hbm.at[gu1_v], du.at[pl.ds(half, half)], s3),
          pltpu.async_copy(Vp_hbm.at[gi1_v], di.at[pl.ds(half, half)], s4),
          pltpu.async_copy(Vp_hbm.at[gj1_v], dj.at[pl.ds(half, half)], s5)]

    def chunk(c, sq):
        cb = c * _LANES
        hu16 = hu_v[pl.ds(cb, _LANES)]
        hi16 = hi_v[pl.ds(cb, _LANES)]
        hj16 = hj_v[pl.ds(cb, _LANES)]
        for t in range(_LANES):
            b = cb + t
            hu = hu16[t]
            hi = hi16[t]
            hj = hj16[t]
            acc = jnp.zeros((_LANES,), jnp.float32)
            for q in range(_D // _LANES):
                fu = du[b, pl.ds(hu + q * _LANES, _LANES)]
                fi_ = di[b, pl.ds(hi + q * _LANES, _LANES)]
                fj_ = dj[b, pl.ds(hj + q * _LANES, _LANES)]
                acc = acc + fu * (fi_ - fj_)
                sq = sq + fu * fu + fi_ * fi_ + fj_ * fj_
            part_v[b, :] = acc
        return sq

    for cp in ga:
        cp.wait()
    sq = lax.fori_loop(0, _BPW // _LANES // 2, chunk,
                       jnp.zeros((_LANES,), jnp.float32))
    for cp in gb:
        cp.wait()
    sq = lax.fori_loop(_BPW // _LANES // 2, _BPW // _LANES, chunk, sq)
    sq_v[...] = sq
    pltpu.sync_copy(part_v, part_hbm.at[pl.ds(base, _BPW)])
    pltpu.sync_copy(sq_v, sq_hbm.at[pl.ds(wid * _LANES, _LANES)])


_sc_call = functools.partial(
    pl.kernel,
    out_type=(
        jax.ShapeDtypeStruct((_B, _LANES), jnp.float32),
        jax.ShapeDtypeStruct((_NW * _LANES,), jnp.float32),
    ),
    mesh=plsc.VectorSubcoreMesh(core_axis_name="c", subcore_axis_name="s"),
    scratch_types=[
        pltpu.VMEM((_BPW,), jnp.int32),
        pltpu.VMEM((_BPW,), jnp.int32),
        pltpu.VMEM((_BPW,), jnp.int32),
        pltpu.VMEM((_BPW,), jnp.int32),
        pltpu.VMEM((_BPW,), jnp.int32),
        pltpu.VMEM((_BPW,), jnp.int32),
        pltpu.VMEM((_BPW // 2,), jnp.int32),
        pltpu.VMEM((_BPW // 2,), jnp.int32),
        pltpu.VMEM((_BPW // 2,), jnp.int32),
        pltpu.VMEM((_BPW // 2,), jnp.int32),
        pltpu.VMEM((_BPW // 2,), jnp.int32),
        pltpu.VMEM((_BPW // 2,), jnp.int32),
        pltpu.VMEM((_BPW, 2 * _D), jnp.float32),
        pltpu.VMEM((_BPW, 2 * _D), jnp.float32),
        pltpu.VMEM((_BPW, 2 * _D), jnp.float32),
        pltpu.VMEM((_BPW, _LANES), jnp.float32),
        pltpu.VMEM((_LANES,), jnp.float32),
        pltpu.SemaphoreType.DMA,
        pltpu.SemaphoreType.DMA,
        pltpu.SemaphoreType.DMA,
        pltpu.SemaphoreType.DMA,
        pltpu.SemaphoreType.DMA,
        pltpu.SemaphoreType.DMA,
    ],
)(_sc_body)


def _tc_body(part_ref, sq_ref, out_ref):
    x = jnp.sum(part_ref[...], axis=1, keepdims=True)
    s = jnp.sum(jnp.log2(jax.nn.sigmoid(x)))
    ssq = jnp.sum(sq_ref[...])
    out_ref[0, 0] = _REG * ssq - s


def kernel(u, i, j, U, V):
    u = u.astype(jnp.int32)
    i = i.astype(jnp.int32)
    j = j.astype(jnp.int32)
    Up, Vp = _pack_tables(U.T, V.T)
    part, sq = _sc_call(u, i, j, Up, Vp)
    loss = pl.pallas_call(
        _tc_body,
        out_shape=jax.ShapeDtypeStruct((1, 1), jnp.float32),
        in_specs=[
            pl.BlockSpec(memory_space=pltpu.VMEM),
            pl.BlockSpec(memory_space=pltpu.VMEM),
        ],
        out_specs=pl.BlockSpec(memory_space=pltpu.SMEM),
    )(part, sq.reshape(4, _NW * _LANES // 4))
    return loss[0, 0]
